# baseline (speedup 1.0000x reference)
import functools

import jax
import jax.numpy as jnp
from jax import lax
from jax.experimental import pallas as pl
from jax.experimental.pallas import tpu as pltpu
from jax.experimental.pallas import tpu_sc as plsc

_CONTEXT = 20
_VOCAB = 100000
_EMBED = 64
_HIDDEN = 100

_NW = 32
_VT = 59136
_VBT = 19712
_NBT = 3
_SPAN = 1152
_CH = 512
_WIDTHS = (512, 512, 128)
_OFFS = (0, 512, 1024)
_TAILW = 6400
_TAIL0 = 15 * _TAILW
_NEG = -1e30


def _hidden_body(idx_ref, *refs):
    emb_refs = refs[:_CONTEXT]
    w1t_ref, b1_ref, h_ref = refs[_CONTEXT], refs[_CONTEXT + 1], refs[_CONTEXT + 2]
    acc = b1_ref[...]
    lane = jax.lax.broadcasted_iota(jnp.int32, (128, 1), 0)
    for t in range(_CONTEXT):
        onehot = (lane == idx_ref[t] % 128).astype(jnp.float32)
        col = jnp.dot(
            emb_refs[t][...], onehot, preferred_element_type=jnp.float32
        )
        acc = acc + jax.lax.dot_general(
            col,
            w1t_ref[:, pl.ds(t * _EMBED, _EMBED)],
            (((0,), (1,)), ((), ())),
            preferred_element_type=jnp.float32,
        )
    h_ref[...] = jnp.tanh(acc)


def _emb_spec(t):
    return pl.BlockSpec((_EMBED, 128), lambda i, idx: (0, idx[t] // 128))


def _hidden(inputs, emb_t, W1t, b1):
    return pl.pallas_call(
        _hidden_body,
        grid_spec=pltpu.PrefetchScalarGridSpec(
            num_scalar_prefetch=1,
            grid=(1,),
            in_specs=[_emb_spec(t) for t in range(_CONTEXT)]
            + [
                pl.BlockSpec(
                    (_HIDDEN, _CONTEXT * _EMBED), lambda i, idx: (0, 0)
                ),
                pl.BlockSpec((1, _HIDDEN), lambda i, idx: (0, 0)),
            ],
            out_specs=pl.BlockSpec((1, _HIDDEN), lambda i, idx: (0, 0)),
        ),
        out_shape=jax.ShapeDtypeStruct((1, _HIDDEN), jnp.float32),
    )(
        inputs.astype(jnp.int32),
        *([emb_t] * _CONTEXT),
        W1t,
        b1.reshape(1, _HIDDEN),
    )


def _sc_body(
    h_hbm,
    w2_hbm,
    b2_hbm,
    logits_hbm,
    m_hbm,
    s_hbm,
    h_vmem,
    b2_vmem,
    logits_vmem,
    stat_vmem,
    w2_a,
    w2_b,
    sem_a,
    sem_b,
    sem_c,
):
    wid = lax.axis_index("s") * 2 + lax.axis_index("c")
    pltpu.sync_copy(h_hbm, h_vmem)
    bufs = (w2_a, w2_b)
    sems = (sem_a, sem_b)

    def fma_chunk(w2_vmem, c, width):
        ng = width // 16

        def fma_body(k4, accs):
            for u in range(4):
                k = k4 * 4 + u
                hv = h_vmem[pl.ds(k * 16, 16)]
                accs = tuple(
                    accs[g] + hv * w2_vmem[k, pl.ds(g * 16, 16)]
                    for g in range(ng)
                )
            return accs

        init = tuple(
            b2_vmem[pl.ds(_OFFS[c] + g * 16, 16)] for g in range(ng)
        )
        return lax.fori_loop(0, _HIDDEN // 4, fma_body, init)

    def stats_update(m_vec, s_vec, accs):
        chunk_m = accs[0]
        for a in accs[1:]:
            chunk_m = jnp.maximum(chunk_m, a)
        m_new = jnp.maximum(m_vec, chunk_m)
        s_new = s_vec * jnp.exp(m_vec - m_new)
        for a in accs:
            s_new = s_new + jnp.exp(a - m_new)
        return m_new, s_new

    def write_stats(m_vec, s_vec):
        stat_vmem[pl.ds(0, 16)] = m_vec
        pltpu.sync_copy(
            stat_vmem.at[pl.ds(0, 16)], m_hbm.at[pl.ds(wid * 16, 16)]
        )
        stat_vmem[pl.ds(16, 16)] = s_vec
        pltpu.sync_copy(
            stat_vmem.at[pl.ds(16, 16)], s_hbm.at[pl.ds(wid * 16, 16)]
        )

    @pl.when(wid < _NW)
    def _main():
        base = _VT + wid * _SPAN
        nch = len(_WIDTHS)
        b2_cp = pltpu.async_copy(
            b2_hbm.at[pl.ds(base, _SPAN)], b2_vmem, sem_c
        )
        pending = [
            pltpu.async_copy(
                w2_hbm.at[:, pl.ds(base, _WIDTHS[0])],
                bufs[0].at[:, pl.ds(0, _WIDTHS[0])],
                sems[0],
            )
        ]
        m_vec = jnp.full((16,), _NEG, jnp.float32)
        s_vec = jnp.zeros((16,), jnp.float32)
        b2_cp.wait()
        for c in range(nch):
            w = _WIDTHS[c]
            if c + 1 < nch:
                pending.append(
                    pltpu.async_copy(
                        w2_hbm.at[:, pl.ds(base + _OFFS[c + 1], _WIDTHS[c + 1])],
                        bufs[(c + 1) % 2].at[:, pl.ds(0, _WIDTHS[c + 1])],
                        sems[(c + 1) % 2],
                    )
                )
            pending.pop(0).wait()
            accs = fma_chunk(bufs[c % 2], c, w)
            for g in range(w // 16):
                logits_vmem[pl.ds(_OFFS[c] + g * 16, 16)] = accs[g]
            m_vec, s_vec = stats_update(m_vec, s_vec, accs)
        pltpu.sync_copy(logits_vmem, logits_hbm.at[pl.ds(base, _SPAN)])
        write_stats(m_vec, s_vec)


def _sc_logits(h, W2, b2):
    mesh = plsc.VectorSubcoreMesh(core_axis_name="c", subcore_axis_name="s")
    run = pl.kernel(
        _sc_body,
        mesh=mesh,
        out_type=[
            jax.ShapeDtypeStruct((_VOCAB,), jnp.float32),
            jax.ShapeDtypeStruct((_NW * 16,), jnp.float32),
            jax.ShapeDtypeStruct((_NW * 16,), jnp.float32),
        ],
        scratch_types=[
            pltpu.VMEM((_HIDDEN * 16,), jnp.float32),
            pltpu.VMEM((_SPAN,), jnp.float32),
            pltpu.VMEM((_SPAN,), jnp.float32),
            pltpu.VMEM((32,), jnp.float32),
            pltpu.VMEM((_HIDDEN, _CH), jnp.float32),
            pltpu.VMEM((_HIDDEN, _CH), jnp.float32),
            pltpu.SemaphoreType.DMA,
            pltpu.SemaphoreType.DMA,
            pltpu.SemaphoreType.DMA,
        ],
    )
    hb = jnp.broadcast_to(h.reshape(_HIDDEN, 1), (_HIDDEN, 16)).reshape(
        _HIDDEN * 16
    )
    return run(hb, W2, b2)


_VB2 = 32768
_NB2 = 4
_LT_BLK = _TAIL0 // _VB2
_LT_BASE = _LT_BLK * _VB2
_LT_LANE = _TAIL0 - _LT_BASE


def _tc_body(
    h_ref, w2p_ref, b2p_ref, w2t_ref, b2t_ref, lp_ref, lt_ref, mt_ref, st_ref,
    stat_ref,
):
    j = pl.program_id(0)
    xp = (
        jnp.dot(h_ref[...], w2p_ref[...], preferred_element_type=jnp.float32)
        + b2p_ref[...]
    )
    lp_ref[...] = xp
    bm = jnp.max(xp)

    @pl.when(j == 0)
    def _():
        xt = (
            jnp.dot(
                h_ref[...], w2t_ref[...], preferred_element_type=jnp.float32
            )
            + b2t_ref[...]
        )
        colt = _TAIL0 + jax.lax.broadcasted_iota(jnp.int32, (1, _TAILW), 1)
        xt = jnp.where(colt < _VOCAB, xt, -jnp.inf)
        lt_ref[pl.ds(0, 1), pl.ds(_LT_LANE, _TAILW)] = xt
        mt = jnp.maximum(jnp.max(xt), bm)
        stat_ref[1] = jnp.sum(jnp.exp(xt - mt)) + jnp.sum(jnp.exp(xp - mt))
        stat_ref[0] = mt

    @pl.when(j > 0)
    def _():
        m_old = stat_ref[0]
        m_new = jnp.maximum(m_old, bm)
        stat_ref[1] = stat_ref[1] * jnp.exp(m_old - m_new) + jnp.sum(
            jnp.exp(xp - m_new)
        )
        stat_ref[0] = m_new

    @pl.when(j == _NBT - 1)
    def _():
        mt_ref[0, 0] = stat_ref[0]
        st_ref[0, 0] = stat_ref[1]


def _tc_logits(h, W2, b2_2d):
    return pl.pallas_call(
        _tc_body,
        grid=(_NBT,),
        in_specs=[
            pl.BlockSpec((1, _HIDDEN), lambda j: (0, 0)),
            pl.BlockSpec((_HIDDEN, _VBT), lambda j: (0, j)),
            pl.BlockSpec((1, _VBT), lambda j: (0, j)),
            pl.BlockSpec((_HIDDEN, _TAILW), lambda j: (0, 15)),
            pl.BlockSpec((1, _TAILW), lambda j: (0, 15)),
        ],
        out_specs=[
            pl.BlockSpec((1, _VBT), lambda j: (0, j)),
            pl.BlockSpec((1, 2 * _VB2), lambda j: (0, 0)),
            pl.BlockSpec(memory_space=pltpu.SMEM),
            pl.BlockSpec(memory_space=pltpu.SMEM),
        ],
        out_shape=[
            jax.ShapeDtypeStruct((1, _VT), jnp.float32),
            jax.ShapeDtypeStruct((1, 2 * _VB2), jnp.float32),
            jax.ShapeDtypeStruct((1, 1), jnp.float32),
            jax.ShapeDtypeStruct((1, 1), jnp.float32),
        ],
        scratch_shapes=[pltpu.SMEM((2,), jnp.float32)],
    )(h, W2, b2_2d, W2, b2_2d)


def _norm_body(
    lp_ref, ls_ref, lt_ref, m_ref, s_ref, mt_ref, st_ref, out_ref, lse_ref
):
    j = pl.program_id(0)

    @pl.when(j == 0)
    def _():
        m_sc = jnp.max(m_ref[...])
        m_g = jnp.maximum(m_sc, mt_ref[0, 0])
        s_g = jnp.sum(s_ref[...] * jnp.exp(m_ref[...] - m_g)) + st_ref[
            0, 0
        ] * jnp.exp(mt_ref[0, 0] - m_g)
        lse_ref[0] = m_g + jnp.log(s_g)

    lse = lse_ref[0]
    col = j * _VB2 + jax.lax.broadcasted_iota(jnp.int32, (1, _VB2), 1)
    v = jnp.where(col < _VT, lp_ref[...], ls_ref[...].reshape(1, _VB2))
    v = jnp.where(col >= _TAIL0, lt_ref[...], v)
    out_ref[...] = v - lse


def _normalize(lp, ls2d, lt, m_part, s_part, mt, st):
    return pl.pallas_call(
        _norm_body,
        grid=(_NB2,),
        in_specs=[
            pl.BlockSpec((1, _VB2), lambda j: (0, jnp.minimum(j, _VT // _VB2))),
            pl.BlockSpec((_VB2,), lambda j: (j,)),
            pl.BlockSpec(
                (1, _VB2), lambda j: (0, jnp.clip(j - _LT_BLK, 0, 1))
            ),
            pl.BlockSpec((1, _NW * 16), lambda j: (0, 0)),
            pl.BlockSpec((1, _NW * 16), lambda j: (0, 0)),
            pl.BlockSpec(memory_space=pltpu.SMEM),
            pl.BlockSpec(memory_space=pltpu.SMEM),
        ],
        out_specs=pl.BlockSpec((1, _VB2), lambda j: (0, j)),
        out_shape=jax.ShapeDtypeStruct((1, _VOCAB), jnp.float32),
        scratch_shapes=[pltpu.SMEM((1,), jnp.float32)],
    )(lp, ls2d, lt, m_part, s_part, mt, st)


def kernel(inputs, emb_table, W1, b1, W2, b2):
    h = _hidden(inputs, emb_table.T, W1.T, b1)
    logits, m_part, s_part = _sc_logits(h, W2, b2)
    lp, lt, mt, st = _tc_logits(h, W2, b2.reshape(1, _VOCAB))
    return _normalize(
        lp,
        logits,
        lt,
        m_part.reshape(1, _NW * 16),
        s_part.reshape(1, _NW * 16),
        mt,
        st,
    )


# speedup vs baseline: 1.0036x; 1.0036x over previous
import jax
import jax.numpy as jnp
from jax import lax
from jax.experimental import pallas as pl
from jax.experimental.pallas import tpu as pltpu
from jax.experimental.pallas import tpu_sc as plsc

_CONTEXT = 20
_VOCAB = 100000
_EMBED = 64
_HIDDEN = 100

_NW = 32
_VT = 59136
_VBT = 19712
_NBT = 3
_SPAN = 1152
_CH = 512
_WIDTHS = (512, 512, 128)
_OFFS = (0, 512, 1024)
_TAILW = 6400
_TAIL0 = 15 * _TAILW
_NEG = -1e30


def _hidden_body(idx_ref, *refs):
    emb_refs = refs[:_CONTEXT]
    w1t_ref, b1_ref, h_ref = refs[_CONTEXT], refs[_CONTEXT + 1], refs[_CONTEXT + 2]
    acc = b1_ref[...]
    lane = jax.lax.broadcasted_iota(jnp.int32, (128, 1), 0)
    for t in range(_CONTEXT):
        onehot = (lane == idx_ref[t] % 128).astype(jnp.float32)
        col = jnp.dot(
            emb_refs[t][...], onehot, preferred_element_type=jnp.float32
        )
        acc = acc + jax.lax.dot_general(
            col,
            w1t_ref[:, pl.ds(t * _EMBED, _EMBED)],
            (((0,), (1,)), ((), ())),
            preferred_element_type=jnp.float32,
        )
    h_ref[...] = jnp.tanh(acc)


def _emb_spec(t):
    return pl.BlockSpec((_EMBED, 128), lambda i, idx: (0, idx[t] // 128))


def _hidden(inputs, emb_t, W1t, b1):
    return pl.pallas_call(
        _hidden_body,
        grid_spec=pltpu.PrefetchScalarGridSpec(
            num_scalar_prefetch=1,
            grid=(1,),
            in_specs=[_emb_spec(t) for t in range(_CONTEXT)]
            + [
                pl.BlockSpec(
                    (_HIDDEN, _CONTEXT * _EMBED), lambda i, idx: (0, 0)
                ),
                pl.BlockSpec((1, _HIDDEN), lambda i, idx: (0, 0)),
            ],
            out_specs=pl.BlockSpec((1, _HIDDEN), lambda i, idx: (0, 0)),
        ),
        out_shape=jax.ShapeDtypeStruct((1, _HIDDEN), jnp.float32),
    )(
        inputs.astype(jnp.int32),
        *([emb_t] * _CONTEXT),
        W1t,
        b1.reshape(1, _HIDDEN),
    )


def _sc_body(
    h_hbm,
    w2_hbm,
    b2_hbm,
    logits_hbm,
    m_hbm,
    s_hbm,
    h_vmem,
    b2_vmem,
    logits_vmem,
    stat_vmem,
    w2_a,
    w2_b,
    sem_a,
    sem_b,
    sem_c,
):
    wid = lax.axis_index("s") * 2 + lax.axis_index("c")
    pltpu.sync_copy(h_hbm, h_vmem)
    bufs = (w2_a, w2_b)
    sems = (sem_a, sem_b)

    def fma_chunk(w2_vmem, c, width):
        ng = width // 16

        def fma_body(k4, accs):
            for u in range(4):
                k = k4 * 4 + u
                hv = h_vmem[pl.ds(k * 16, 16)]
                accs = tuple(
                    accs[g] + hv * w2_vmem[k, pl.ds(g * 16, 16)]
                    for g in range(ng)
                )
            return accs

        init = tuple(
            b2_vmem[pl.ds(_OFFS[c] + g * 16, 16)] for g in range(ng)
        )
        return lax.fori_loop(0, _HIDDEN // 4, fma_body, init)

    def stats_update(m_vec, s_vec, accs):
        chunk_m = accs[0]
        for a in accs[1:]:
            chunk_m = jnp.maximum(chunk_m, a)
        m_new = jnp.maximum(m_vec, chunk_m)
        s_new = s_vec * jnp.exp(m_vec - m_new)
        for a in accs:
            s_new = s_new + jnp.exp(a - m_new)
        return m_new, s_new

    def write_stats(m_vec, s_vec):
        stat_vmem[pl.ds(0, 16)] = m_vec
        pltpu.sync_copy(
            stat_vmem.at[pl.ds(0, 16)], m_hbm.at[pl.ds(wid * 16, 16)]
        )
        stat_vmem[pl.ds(16, 16)] = s_vec
        pltpu.sync_copy(
            stat_vmem.at[pl.ds(16, 16)], s_hbm.at[pl.ds(wid * 16, 16)]
        )

    @pl.when(wid < _NW)
    def _main():
        base = _VT + wid * _SPAN
        nch = len(_WIDTHS)
        b2_cp = pltpu.async_copy(
            b2_hbm.at[pl.ds(base, _SPAN)], b2_vmem, sem_c
        )
        pending = [
            pltpu.async_copy(
                w2_hbm.at[:, pl.ds(base, _WIDTHS[0])],
                bufs[0].at[:, pl.ds(0, _WIDTHS[0])],
                sems[0],
            )
        ]
        m_vec = jnp.full((16,), _NEG, jnp.float32)
        s_vec = jnp.zeros((16,), jnp.float32)
        b2_cp.wait()
        for c in range(nch):
            w = _WIDTHS[c]
            if c + 1 < nch:
                pending.append(
                    pltpu.async_copy(
                        w2_hbm.at[:, pl.ds(base + _OFFS[c + 1], _WIDTHS[c + 1])],
                        bufs[(c + 1) % 2].at[:, pl.ds(0, _WIDTHS[c + 1])],
                        sems[(c + 1) % 2],
                    )
                )
            pending.pop(0).wait()
            accs = fma_chunk(bufs[c % 2], c, w)
            for g in range(w // 16):
                logits_vmem[pl.ds(_OFFS[c] + g * 16, 16)] = accs[g]
            m_vec, s_vec = stats_update(m_vec, s_vec, accs)
        pltpu.sync_copy(logits_vmem, logits_hbm.at[pl.ds(base, _SPAN)])
        write_stats(m_vec, s_vec)


def _sc_logits(h, W2, b2):
    mesh = plsc.VectorSubcoreMesh(core_axis_name="c", subcore_axis_name="s")
    run = pl.kernel(
        _sc_body,
        mesh=mesh,
        out_type=[
            jax.ShapeDtypeStruct((_VOCAB,), jnp.float32),
            jax.ShapeDtypeStruct((_NW * 16,), jnp.float32),
            jax.ShapeDtypeStruct((_NW * 16,), jnp.float32),
        ],
        scratch_types=[
            pltpu.VMEM((_HIDDEN * 16,), jnp.float32),
            pltpu.VMEM((_SPAN,), jnp.float32),
            pltpu.VMEM((_SPAN,), jnp.float32),
            pltpu.VMEM((32,), jnp.float32),
            pltpu.VMEM((_HIDDEN, _CH), jnp.float32),
            pltpu.VMEM((_HIDDEN, _CH), jnp.float32),
            pltpu.SemaphoreType.DMA,
            pltpu.SemaphoreType.DMA,
            pltpu.SemaphoreType.DMA,
        ],
    )
    hb = jnp.broadcast_to(h.reshape(_HIDDEN, 1), (_HIDDEN, 16)).reshape(
        _HIDDEN * 16
    )
    return run(hb, W2, b2)


_VB2 = 32768
_NB2 = 4
_LT_BLK = _TAIL0 // _VB2
_LT_BASE = _LT_BLK * _VB2
_LT_LANE = _TAIL0 - _LT_BASE


def _tc_body(
    h_ref, w2p_ref, b2p_ref, w2t_ref, b2t_ref, lp_ref, lt_ref, mt_ref, st_ref,
    stat_ref,
):
    j = pl.program_id(0)
    xp = (
        jnp.dot(h_ref[...], w2p_ref[...], preferred_element_type=jnp.float32)
        + b2p_ref[...]
    )
    lp_ref[...] = xp
    bm = jnp.max(xp)

    @pl.when(j == 0)
    def _():
        xt = (
            jnp.dot(
                h_ref[...], w2t_ref[...], preferred_element_type=jnp.float32
            )
            + b2t_ref[...]
        )
        colt = _TAIL0 + jax.lax.broadcasted_iota(jnp.int32, (1, _TAILW), 1)
        xt = jnp.where(colt < _VOCAB, xt, -jnp.inf)
        lt_ref[pl.ds(0, 1), pl.ds(_LT_LANE, _TAILW)] = xt
        mt = jnp.maximum(jnp.max(xt), bm)
        stat_ref[1] = jnp.sum(jnp.exp(xt - mt)) + jnp.sum(jnp.exp(xp - mt))
        stat_ref[0] = mt

    @pl.when(j > 0)
    def _():
        m_old = stat_ref[0]
        m_new = jnp.maximum(m_old, bm)
        stat_ref[1] = stat_ref[1] * jnp.exp(m_old - m_new) + jnp.sum(
            jnp.exp(xp - m_new)
        )
        stat_ref[0] = m_new

    @pl.when(j == _NBT - 1)
    def _():
        mt_ref[0, 0] = stat_ref[0]
        st_ref[0, 0] = stat_ref[1]


def _tc_logits(h, W2, b2_2d):
    return pl.pallas_call(
        _tc_body,
        grid=(_NBT,),
        in_specs=[
            pl.BlockSpec((1, _HIDDEN), lambda j: (0, 0)),
            pl.BlockSpec((_HIDDEN, _VBT), lambda j: (0, j)),
            pl.BlockSpec((1, _VBT), lambda j: (0, j)),
            pl.BlockSpec((_HIDDEN, _TAILW), lambda j: (0, 15)),
            pl.BlockSpec((1, _TAILW), lambda j: (0, 15)),
        ],
        out_specs=[
            pl.BlockSpec((1, _VBT), lambda j: (0, j)),
            pl.BlockSpec((1, 2 * _VB2), lambda j: (0, 0)),
            pl.BlockSpec(memory_space=pltpu.SMEM),
            pl.BlockSpec(memory_space=pltpu.SMEM),
        ],
        out_shape=[
            jax.ShapeDtypeStruct((1, _VT), jnp.float32),
            jax.ShapeDtypeStruct((1, 2 * _VB2), jnp.float32),
            jax.ShapeDtypeStruct((1, 1), jnp.float32),
            jax.ShapeDtypeStruct((1, 1), jnp.float32),
        ],
        scratch_shapes=[pltpu.SMEM((2,), jnp.float32)],
    )(h, W2, b2_2d, W2, b2_2d)


def _norm_body(
    lp_ref, ls_ref, lt_ref, m_ref, s_ref, mt_ref, st_ref, out_ref, lse_ref
):
    j = pl.program_id(0)

    @pl.when(j == 0)
    def _():
        m_sc = jnp.max(m_ref[...])
        m_g = jnp.maximum(m_sc, mt_ref[0, 0])
        s_g = jnp.sum(s_ref[...] * jnp.exp(m_ref[...] - m_g)) + st_ref[
            0, 0
        ] * jnp.exp(mt_ref[0, 0] - m_g)
        lse_ref[0] = m_g + jnp.log(s_g)

    lse = lse_ref[0]
    col = j * _VB2 + jax.lax.broadcasted_iota(jnp.int32, (1, _VB2), 1)
    v = jnp.where(col < _VT, lp_ref[...], ls_ref[...].reshape(1, _VB2))
    v = jnp.where(col >= _TAIL0, lt_ref[...], v)
    out_ref[...] = v - lse


def _normalize(lp, ls2d, lt, m_part, s_part, mt, st):
    return pl.pallas_call(
        _norm_body,
        grid=(_NB2,),
        in_specs=[
            pl.BlockSpec((1, _VB2), lambda j: (0, jnp.minimum(j, _VT // _VB2))),
            pl.BlockSpec((_VB2,), lambda j: (j,)),
            pl.BlockSpec(
                (1, _VB2), lambda j: (0, jnp.clip(j - _LT_BLK, 0, 1))
            ),
            pl.BlockSpec((1, _NW * 16), lambda j: (0, 0)),
            pl.BlockSpec((1, _NW * 16), lambda j: (0, 0)),
            pl.BlockSpec(memory_space=pltpu.SMEM),
            pl.BlockSpec(memory_space=pltpu.SMEM),
        ],
        out_specs=pl.BlockSpec((1, _VB2), lambda j: (0, j)),
        out_shape=jax.ShapeDtypeStruct((1, _VOCAB), jnp.float32),
        scratch_shapes=[pltpu.SMEM((1,), jnp.float32)],
    )(lp, ls2d, lt, m_part, s_part, mt, st)


def kernel(inputs, emb_table, W1, b1, W2, b2):
    h = _hidden(inputs, emb_table.T, W1.T, b1)
    logits, m_part, s_part = _sc_logits(h, W2, b2)
    lp, lt, mt, st = _tc_logits(h, W2, b2.reshape(1, _VOCAB))
    return _normalize(
        lp,
        logits,
        lt,
        m_part.reshape(1, _NW * 16),
        s_part.reshape(1, _NW * 16),
        mt,
        st,
    )
